# baseline (speedup 1.0000x reference)
import jax
import jax.numpy as jnp
from jax import lax
from jax.experimental import pallas as pl
from jax.experimental.pallas import tpu as pltpu

HIDDEN = 2048
N_EXP = 8
BLKM = 1024
NBUF = 4
NOUT = 2


def _tc_kernel(x_hbm, w_ref, o_hbm, xbufs, obufs, insem, outsem):
    nblk = x_hbm.shape[0] // BLKM

    def in_start(b, buf, prio):
        pltpu.make_async_copy(
            x_hbm.at[pl.ds(b * BLKM, BLKM)], xbufs.at[buf],
            insem.at[buf]).start(priority=prio)

    def in_wait(buf):
        pltpu.make_async_copy(
            x_hbm.at[pl.ds(0, BLKM)], xbufs.at[buf], insem.at[buf]).wait()

    def out_copy(b, obuf):
        return pltpu.make_async_copy(
            obufs.at[obuf], o_hbm.at[pl.ds(b * BLKM, BLKM)], outsem.at[obuf])

    for i in range(NBUF):
        in_start(i, i, i % 2)

    @pl.loop(0, nblk, step=2)
    def _(b):
        for sub in range(2):
            bb = b + sub
            buf = lax.rem(bb, NBUF)
            obuf = lax.rem(bb, NOUT)
            in_wait(buf)

            @pl.when(bb >= NOUT)
            def _():
                out_copy(bb - NOUT, obuf).wait()

            obufs[obuf] = jax.lax.dot_general(
                xbufs[buf], w_ref[...],
                dimension_numbers=(((1,), (1,)), ((), ())),
                preferred_element_type=jnp.float32,
            )
            out_copy(bb, obuf).start()

            @pl.when(bb + NBUF < nblk)
            def _():
                in_start(bb + NBUF, buf, sub)

    for i in range(NOUT):
        out_copy(nblk - NOUT + i, lax.rem(nblk - NOUT + i, NOUT)).wait()


def kernel(x, weight):
    xf = x.reshape(-1, HIDDEN)
    rows = xf.shape[0]
    out = pl.pallas_call(
        _tc_kernel,
        in_specs=[
            pl.BlockSpec(memory_space=pl.MemorySpace.ANY),
            pl.BlockSpec((N_EXP, HIDDEN), lambda: (0, 0)),
        ],
        out_specs=pl.BlockSpec(memory_space=pl.MemorySpace.ANY),
        out_shape=jax.ShapeDtypeStruct((rows, N_EXP), jnp.float32),
        scratch_shapes=[
            pltpu.VMEM((NBUF, BLKM, HIDDEN), jnp.float32),
            pltpu.VMEM((NOUT, BLKM, N_EXP), jnp.float32),
            pltpu.SemaphoreType.DMA((NBUF,)),
            pltpu.SemaphoreType.DMA((NOUT,)),
        ],
    )(xf, weight)
    return out
